# Initial kernel scaffold; baseline (speedup 1.0000x reference)
import jax
import jax.numpy as jnp
from jax.experimental import pallas as pl


def kernel(inputs, table, W1, b1, W2, b2):
    raise NotImplementedError("write your pallas kernel here")


# speedup vs baseline: 90.8442x; 90.8442x over previous
import functools

import jax
import jax.numpy as jnp
from jax import lax
from jax.experimental import pallas as pl
from jax.experimental.pallas import tpu as pltpu
from jax.experimental.pallas import tpu_sc as plsc

_VOCAB = 1000
_EMB = 64
_FULL = 5
_B = 4096
_L = 50
_LW = _L - _FULL + 1
_VPAD = 1024
_PCOLS = 8
_LPAD = 64
_TPAD = 48

_NC = 2
_NS = 16
_ROWS = _B // (_NC * _NS)


def _score_table_kernel(tab_ref, w1_ref, b1_ref, w2_ref, b2_ref, p_ref):
    tab = tab_ref[...]
    w2 = w2_ref[...]
    w1 = w1_ref[...]
    c = jnp.sum(w2 * b1_ref[...]) + b2_ref[0, 0]
    ws = [jnp.dot(w2, w1[:, _EMB * j:_EMB * (j + 1)]) for j in range(_FULL)]
    ws.append(jnp.zeros((_PCOLS - _FULL, _EMB), jnp.float32))
    wstack = jnp.concatenate(ws, axis=0)
    p0 = lax.dot_general(tab, wstack, (((1,), (1,)), ((), ())))
    col = lax.broadcasted_iota(jnp.int32, (_VPAD, _PCOLS), 1)
    p_ref[...] = p0 + jnp.where(col == 0, c, 0.0)


def _window_score_kernel(inp_hbm, p_hbm, out_hbm, inp_v, p_v, out_v):
    wid = lax.axis_index("s") * _NC + lax.axis_index("c")
    base = wid * _ROWS
    pltpu.sync_copy(inp_hbm.at[pl.ds(base, _ROWS)], inp_v)
    pltpu.sync_copy(p_hbm, p_v)

    def body(b, carry):
        for t0 in (0, 16, 32):
            acc = jnp.zeros((16,), jnp.float32)
            for j in range(_FULL):
                idx = inp_v[b, pl.ds(t0 + j, 16)]
                acc = acc + plsc.load_gather(p_v, [idx * _PCOLS + j])
            out_v[b, pl.ds(t0, 16)] = 1.0 / (1.0 + jnp.exp(-acc))
        return carry

    lax.fori_loop(0, _ROWS, body, 0)
    pltpu.sync_copy(out_v, out_hbm.at[pl.ds(base, _ROWS)])


def kernel(inputs, table, W1, b1, W2, b2):
    tab_pad = jnp.zeros((_VPAD, _EMB), jnp.float32).at[:_VOCAB].set(table)
    p = pl.pallas_call(
        _score_table_kernel,
        out_shape=jax.ShapeDtypeStruct((_VPAD, _PCOLS), jnp.float32),
    )(tab_pad, W1, b1.reshape(1, -1), W2, b2.reshape(1, 1))

    inp_pad = jnp.zeros((_B, _LPAD), jnp.int32).at[:, :_L].set(
        inputs.astype(jnp.int32))

    sc = pl.kernel(
        _window_score_kernel,
        out_type=jax.ShapeDtypeStruct((_B, _TPAD), jnp.float32),
        mesh=plsc.VectorSubcoreMesh(core_axis_name="c", subcore_axis_name="s"),
        compiler_params=pltpu.CompilerParams(needs_layout_passes=False),
        scratch_types=[
            pltpu.VMEM((_ROWS, _LPAD), jnp.int32),
            pltpu.VMEM((_VPAD * _PCOLS,), jnp.float32),
            pltpu.VMEM((_ROWS, _TPAD), jnp.float32),
        ],
    )
    out_pad = sc(inp_pad, p.reshape(-1))
    return out_pad[:, :_LW]


# speedup vs baseline: 112.0920x; 1.2339x over previous
import functools

import jax
import jax.numpy as jnp
from jax import lax
from jax.experimental import pallas as pl
from jax.experimental.pallas import tpu as pltpu
from jax.experimental.pallas import tpu_sc as plsc

_VOCAB = 1000
_EMB = 64
_FULL = 5
_B = 4096
_L = 50
_LW = _L - _FULL + 1
_VPAD = 1024
_PCOLS = 8
_LPAD = 64
_TPAD = 48

_NC = 2
_NS = 16
_ROWS = _B // (_NC * _NS)


def _score_table_kernel(tab_ref, w1_ref, b1_ref, w2_ref, b2_ref, p_ref):
    tab = tab_ref[...]
    w2 = w2_ref[...]
    w1 = w1_ref[...]
    c = jnp.sum(w2 * b1_ref[...]) + b2_ref[0, 0]
    ws = [jnp.dot(w2, w1[:, _EMB * j:_EMB * (j + 1)]) for j in range(_FULL)]
    wstack = jnp.concatenate(ws, axis=0)
    p0 = lax.dot_general(wstack, tab, (((1,), (1,)), ((), ())))
    row = lax.broadcasted_iota(jnp.int32, (_FULL, _VPAD), 0)
    p_ref[...] = p0 + jnp.where(row == 0, c, 0.0)


def _window_score_kernel(inp_hbm, p_hbm, out_hbm, inp_v,
                         p0_v, p1_v, p2_v, p3_v, p4_v, out_v, sem):
    wid = lax.axis_index("s") * _NC + lax.axis_index("c")
    base = wid * _ROWS
    p_refs = (p0_v, p1_v, p2_v, p3_v, p4_v)
    copies = [pltpu.async_copy(inp_hbm.at[pl.ds(base, _ROWS)], inp_v, sem)]
    copies += [pltpu.async_copy(p_hbm.at[pl.ds(j * _VPAD, _VPAD)], p_refs[j], sem)
               for j in range(_FULL)]
    for cp in copies:
        cp.wait()

    @plsc.parallel_loop(0, _ROWS, 1, unroll=4)
    def body(b):
        for t0 in (0, 16, 32):
            g = [plsc.load_gather(p_refs[j], [inp_v[b, pl.ds(t0 + j, 16)]])
                 for j in range(_FULL)]
            acc = ((g[0] + g[1]) + (g[2] + g[3])) + g[4]
            out_v[b, pl.ds(t0, 16)] = 1.0 / (1.0 + jnp.exp(-acc))

    pltpu.sync_copy(out_v, out_hbm.at[pl.ds(base, _ROWS)])


def kernel(inputs, table, W1, b1, W2, b2):
    tab_pad = jnp.zeros((_VPAD, _EMB), jnp.float32).at[:_VOCAB].set(table)
    p = pl.pallas_call(
        _score_table_kernel,
        out_shape=jax.ShapeDtypeStruct((_FULL, _VPAD), jnp.float32),
    )(tab_pad, W1, b1.reshape(1, -1), W2, b2.reshape(1, 1))

    inp_pad = jnp.zeros((_B, _LPAD), jnp.int32).at[:, :_L].set(
        inputs.astype(jnp.int32))

    sc = pl.kernel(
        _window_score_kernel,
        out_type=jax.ShapeDtypeStruct((_B, _TPAD), jnp.float32),
        mesh=plsc.VectorSubcoreMesh(core_axis_name="c", subcore_axis_name="s"),
        compiler_params=pltpu.CompilerParams(needs_layout_passes=False),
        scratch_types=[
            pltpu.VMEM((_ROWS, _LPAD), jnp.int32),
            pltpu.VMEM((_VPAD,), jnp.float32),
            pltpu.VMEM((_VPAD,), jnp.float32),
            pltpu.VMEM((_VPAD,), jnp.float32),
            pltpu.VMEM((_VPAD,), jnp.float32),
            pltpu.VMEM((_VPAD,), jnp.float32),
            pltpu.VMEM((_ROWS, _TPAD), jnp.float32),
            pltpu.SemaphoreType.DMA,
        ],
    )
    out_pad = sc(inp_pad, p.reshape(-1))
    return out_pad[:, :_LW]


# speedup vs baseline: 114.8907x; 1.0250x over previous
import jax
import jax.numpy as jnp
from jax import lax
from jax.experimental import pallas as pl
from jax.experimental.pallas import tpu as pltpu
from jax.experimental.pallas import tpu_sc as plsc

_VOCAB = 1000
_EMB = 64
_FULL = 5
_B = 4096
_L = 50
_LW = _L - _FULL + 1
_VPAD = 1024
_LPAD = 64
_TPAD = 48

_NC = 2
_NS = 16
_ROWS = _B // (_NC * _NS)


def _score_table_kernel(inp_ref, tab_ref, w1_ref, b1_ref, w2_ref, b2_ref,
                        p_ref, ipad_ref):
    tab = tab_ref[...]
    w2 = w2_ref[...]
    w1 = w1_ref[...]
    c = jnp.sum(w2 * b1_ref[...]) + b2_ref[0, 0]
    ws = [jnp.dot(w2, w1[:, _EMB * j:_EMB * (j + 1)]) for j in range(_FULL)]
    wstack = jnp.concatenate(ws, axis=0)
    p0 = lax.dot_general(wstack, tab, (((1,), (1,)), ((), ())))
    row = lax.broadcasted_iota(jnp.int32, (_FULL, _VOCAB), 0)
    p_pad = jnp.pad(p0 + jnp.where(row == 0, c, 0.0),
                    ((0, 0), (0, _VPAD - _VOCAB)))
    p_ref[...] = p_pad.reshape(_FULL * _VPAD // 128, 128)
    ipad_ref[:, :_L] = inp_ref[...]
    ipad_ref[:, _L:] = jnp.zeros((_B, _LPAD - _L), jnp.int32)


def _window_score_kernel(inp_hbm, p_hbm, out_hbm, inp_v,
                         p0_v, p1_v, p2_v, p3_v, p4_v, out_v, sem):
    wid = lax.axis_index("s") * _NC + lax.axis_index("c")
    base = wid * _ROWS
    p_refs = (p0_v, p1_v, p2_v, p3_v, p4_v)
    copies = [pltpu.async_copy(inp_hbm.at[pl.ds(base, _ROWS)], inp_v, sem)]
    copies += [pltpu.async_copy(p_hbm.at[pl.ds(j * _VPAD, _VPAD)], p_refs[j],
                                sem)
               for j in range(_FULL)]
    for cp in copies:
        cp.wait()

    @plsc.parallel_loop(0, _ROWS, 1, unroll=4)
    def body(b):
        for t0 in (0, 16, 32):
            g = [plsc.load_gather(p_refs[j], [inp_v[b, pl.ds(t0 + j, 16)]])
                 for j in range(_FULL)]
            acc = ((g[0] + g[1]) + (g[2] + g[3])) + g[4]
            out_v[b, pl.ds(t0, 16)] = 1.0 / (1.0 + jnp.exp(-acc))

    pltpu.sync_copy(out_v, out_hbm.at[pl.ds(base, _ROWS)])


def kernel(inputs, table, W1, b1, W2, b2):
    p, inp_pad = pl.pallas_call(
        _score_table_kernel,
        out_shape=(
            jax.ShapeDtypeStruct((_FULL * _VPAD // 128, 128), jnp.float32),
            jax.ShapeDtypeStruct((_B, _LPAD), jnp.int32),
        ),
    )(inputs.astype(jnp.int32), table, W1, b1.reshape(1, -1), W2,
      b2.reshape(1, 1))

    sc = pl.kernel(
        _window_score_kernel,
        out_type=jax.ShapeDtypeStruct((_B, _TPAD), jnp.float32),
        mesh=plsc.VectorSubcoreMesh(core_axis_name="c", subcore_axis_name="s"),
        compiler_params=pltpu.CompilerParams(needs_layout_passes=False),
        scratch_types=[
            pltpu.VMEM((_ROWS, _LPAD), jnp.int32),
            pltpu.VMEM((_VPAD,), jnp.float32),
            pltpu.VMEM((_VPAD,), jnp.float32),
            pltpu.VMEM((_VPAD,), jnp.float32),
            pltpu.VMEM((_VPAD,), jnp.float32),
            pltpu.VMEM((_VPAD,), jnp.float32),
            pltpu.VMEM((_ROWS, _TPAD), jnp.float32),
            pltpu.SemaphoreType.DMA,
        ],
    )
    return sc(inp_pad, p.reshape(-1))[:, :_LW]
